# baseline (speedup 1.0000x reference)
import jax
import jax.numpy as jnp
from jax.experimental import pallas as pl

E = 100000
H = 256
BLOCK_ROWS = 2000


def _body(x_ref, y_ref, t_ref, o_ref, g_ref):
    x = x_ref[...]
    y = y_ref[...]
    g = jax.nn.sigmoid(t_ref[...])
    g_ref[...] = g
    o_ref[...] = y + g * (x - y)


def kernel(X, Y, gate_theta):
    grid = (E // BLOCK_ROWS,)
    spec = pl.BlockSpec((BLOCK_ROWS, H), lambda i: (i, 0))
    out, gate = pl.pallas_call(
        _body,
        grid=grid,
        in_specs=[spec, spec, spec],
        out_specs=[spec, spec],
        out_shape=[
            jax.ShapeDtypeStruct((E, H), jnp.float32),
            jax.ShapeDtypeStruct((E, H), jnp.float32),
        ],
    )(X, Y, gate_theta)
    return (out, gate)
